# baseline (speedup 1.0000x reference)
import functools

import jax
import jax.numpy as jnp
from jax import lax
from jax.experimental import pallas as pl
from jax.experimental.pallas import tpu as pltpu
from jax.experimental.pallas import tpu_sc as plsc

B, S, H = 4, 8192, 768
K = S // 2
HQ = H // 4
BLK = 512
NBLK = (B * S) // BLK

NC, NS, L = 2, 16, 16
TPB = (NC * NS) // B
ROWS_PER_TILE = K // TPB
GCH = 64


def _f(v):
    return jnp.float32(v)


def _xla_erfc(x):
    one = _f(1.0)
    ax = jnp.abs(x)
    lt1 = ax < one
    x2 = x * x
    p = x2 * _f(7.85386146e-05)
    p = p + _f(-0.000801019371)
    p = p * x2 + _f(0.00518832775)
    p = p * x2 + _f(-0.0268538129)
    p = p * x2 + _f(0.112835854)
    p = p * x2 + _f(-0.37612626)
    p = p * x2 + _f(1.12837911)
    erf_branch = one - x * p

    nx2 = -x2
    underflow = nx2 < _f(-88.7228394)
    z = jnp.exp(nx2)
    zq = z * (one / ax)
    r = one / x2
    p1 = r * _f(0.0232682)
    p1 = p1 + _f(-0.138703942)
    p1 = p1 * r + _f(0.368742466)
    p1 = p1 * r + _f(-0.582473278)
    p1 = p1 * r + _f(0.621000469)
    p1 = p1 * r + _f(-0.494451523)
    p1 = p1 * r + _f(0.340488)
    p1 = p1 * r + _f(-0.274112701)
    p1 = p1 * r + _f(0.563825965)
    p2 = r * _f(-10.477664)
    p2 = p2 + _f(12.9772)
    p2 = p2 * r + _f(-7.49551868)
    p2 = p2 * r + _f(2.92101908)
    p2 = p2 * r + _f(-1.01526523)
    p2 = p2 * r + _f(0.42184633)
    p2 = p2 * r + _f(-0.282076746)
    p2 = p2 * r + _f(0.564189494)
    poly = jnp.where(ax < _f(2.0), p1, p2)
    y = zq * poly
    y = jnp.where(underflow, _f(0.0), y)
    y = jnp.where(x < _f(0.0), _f(2.0) - y, y)
    return jnp.where(lt1, erf_branch, y)


def _xla_gelu(x):
    return (_f(0.5) * x) * _xla_erfc((-x) * _f(0.7071067690849304))


def _score_body(x_ref, w1_ref, b1_ref, w2_ref, b2_ref, out_ref):
    HB = BLK // 2
    w1 = w1_ref[...]
    w2 = w2_ref[...]
    for t in range(2):
        xb = x_ref[pl.ds(t * HB, HB), :]
        h = jnp.dot(xb, w1, preferred_element_type=jnp.float32)
        h = h + b1_ref[0:1, :]
        g = _xla_gelu(h)
        s = jnp.dot(g, w2, preferred_element_type=jnp.float32)
        s = s[:, 0] + b2_ref[0, 0]
        bits = lax.bitcast_convert_type(s, jnp.uint32)
        key = jnp.where(s >= 0.0, bits | jnp.uint32(0x80000000), ~bits)
        out_ref[0, 0, pl.ds(t * HB, HB)] = key


def _select_body(keys_ref, t_ref, need_ref):
    ku = keys_ref[...]
    p = jnp.zeros((B, 1), jnp.uint32)
    for j in range(31, -1, -1):
        cand = p | (jnp.uint32(1) << j)
        cnt = jnp.sum((ku >= cand).astype(jnp.int32), axis=1, keepdims=True)
        p = jnp.where(cnt >= K, cand, p)
    cnt_gt = jnp.sum((ku > p).astype(jnp.int32), axis=1, keepdims=True)
    need = (K - cnt_gt).astype(jnp.int32)
    pad = jnp.zeros((8 - B, 1), p.dtype)
    t_ref[...] = jnp.broadcast_to(jnp.concatenate([p, pad], axis=0), (8, 128))
    padn = jnp.zeros((8 - B, 1), need.dtype)
    need_ref[...] = jnp.broadcast_to(
        jnp.concatenate([need, padn], axis=0), (8, 128))


def _sc_body(x_hbm, keys_hbm, t_hbm, need_hbm, out_hbm, idx_hbm,
             keys_v, t_v, need_v, idxs_v, my_v, rows_v, sem):
    cid = lax.axis_index("c")
    sid = lax.axis_index("s")
    wid = cid * NS + sid
    b = wid // TPB

    @pl.when(wid % TPB == 0)
    def _select():
        pltpu.sync_copy(keys_hbm.at[pl.ds(b * S, S)], keys_v)
        pltpu.sync_copy(t_hbm.at[pl.ds(b * 128, L)], t_v)
        pltpu.sync_copy(need_hbm.at[pl.ds(b * 128, L)], need_v)
        tv = t_v[...]
        nv = need_v[...]
        base = jnp.int32(b * S)

        def step(i, carry):
            off, eq_seen = carry
            kv = keys_v[pl.ds(i * L, L)]
            gt = kv > tv
            eq = kv == tv
            eqi = jnp.where(eq, jnp.int32(1), jnp.int32(0))
            ecs = plsc.cumsum(eqi)
            keep_eq = jnp.logical_and(eq, (ecs + eq_seen) <= nv)
            m = jnp.logical_or(gt, keep_eq)
            mi = jnp.where(m, jnp.int32(1), jnp.int32(0))
            mcs = plsc.cumsum(mi)
            pos = off + (mcs - mi)
            idxv = lax.iota(jnp.int32, L) + (i * L + base)
            plsc.store_scatter(idxs_v, [pos], idxv, mask=m)
            return off + jnp.max(mcs), eq_seen + jnp.max(ecs)

        lax.fori_loop(0, S // L, step, (jnp.int32(0), jnp.int32(0)))
        pltpu.sync_copy(idxs_v.at[pl.ds(0, K)], idx_hbm.at[pl.ds(b * K, K)])

    plsc.subcore_barrier()

    lo = (wid % TPB) * ROWS_PER_TILE
    pltpu.sync_copy(idx_hbm.at[pl.ds(b * K + lo, ROWS_PER_TILE)], my_v)
    for j in range(ROWS_PER_TILE // GCH):
        idx_slice = my_v.at[pl.ds(j * GCH, GCH)]
        pltpu.async_copy(x_hbm.at[idx_slice], rows_v, sem).wait()
        pltpu.sync_copy(
            rows_v, out_hbm.at[pl.ds(b * K + lo + j * GCH, GCH), :])


def _compact_gather(x2d, keys_flat, t_flat, need_flat):
    mesh = plsc.VectorSubcoreMesh(core_axis_name="c", subcore_axis_name="s")
    kern = pl.kernel(
        _sc_body,
        out_type=(
            jax.ShapeDtypeStruct((B * K, H), jnp.float32),
            jax.ShapeDtypeStruct((B * K,), jnp.int32),
        ),
        mesh=mesh,
        compiler_params=pltpu.CompilerParams(needs_layout_passes=False),
        scratch_types=[
            pltpu.VMEM((S,), jnp.uint32),
            pltpu.VMEM((L,), jnp.uint32),
            pltpu.VMEM((L,), jnp.int32),
            pltpu.VMEM((K + L,), jnp.int32),
            pltpu.VMEM((ROWS_PER_TILE,), jnp.int32),
            pltpu.VMEM((GCH, H), jnp.float32),
            pltpu.SemaphoreType.DMA,
        ],
    )
    return kern(x2d, keys_flat, t_flat, need_flat)


@jax.jit
def kernel(x, W1, b1, W2, b2):
    x2d = x.reshape(B * S, H)
    b1r = jnp.broadcast_to(b1[None, :], (8, HQ))
    w2p = jnp.pad(W2, ((0, 0), (0, 128 - W2.shape[1])))
    b2r = jnp.broadcast_to(b2[None, :], (8, 128))

    keys3 = pl.pallas_call(
        _score_body,
        grid=(NBLK,),
        in_specs=[
            pl.BlockSpec((BLK, H), lambda i: (i, 0)),
            pl.BlockSpec((H, HQ), lambda i: (0, 0)),
            pl.BlockSpec((8, HQ), lambda i: (0, 0)),
            pl.BlockSpec((HQ, 128), lambda i: (0, 0)),
            pl.BlockSpec((8, 128), lambda i: (0, 0)),
        ],
        out_specs=pl.BlockSpec((1, 1, BLK), lambda i: (i, 0, 0)),
        out_shape=jax.ShapeDtypeStruct((NBLK, 1, BLK), jnp.uint32),
    )(x2d, W1, b1r, w2p, b2r)
    keys = keys3.reshape(B, S)

    t8, need8 = pl.pallas_call(
        _select_body,
        out_shape=(
            jax.ShapeDtypeStruct((8, 128), jnp.uint32),
            jax.ShapeDtypeStruct((8, 128), jnp.int32),
        ),
    )(keys)

    out_flat, _ = _compact_gather(
        x2d, keys.reshape(-1), t8.reshape(-1), need8.reshape(-1))
    return out_flat.reshape(B, K, H)


# speedup vs baseline: 1.0207x; 1.0207x over previous
import functools

import jax
import jax.numpy as jnp
from jax import lax
from jax.experimental import pallas as pl
from jax.experimental.pallas import tpu as pltpu
from jax.experimental.pallas import tpu_sc as plsc

B, S, H = 4, 8192, 768
K = S // 2
HQ = H // 4
BLK = 512
NBLK = (B * S) // BLK

NC, NS, L = 2, 16, 16
TPB = (NC * NS) // B
ROWS_PER_TILE = K // TPB
GCH = 64


def _f(v):
    return jnp.float32(v)


def _xla_erfc(x):
    one = _f(1.0)
    ax = jnp.abs(x)
    lt1 = ax < one
    x2 = x * x
    p = x2 * _f(7.85386146e-05)
    p = p + _f(-0.000801019371)
    p = p * x2 + _f(0.00518832775)
    p = p * x2 + _f(-0.0268538129)
    p = p * x2 + _f(0.112835854)
    p = p * x2 + _f(-0.37612626)
    p = p * x2 + _f(1.12837911)
    erf_branch = one - x * p

    nx2 = -x2
    underflow = nx2 < _f(-88.7228394)
    z = jnp.exp(nx2)
    zq = z * (one / ax)
    r = one / x2
    p1 = r * _f(0.0232682)
    p1 = p1 + _f(-0.138703942)
    p1 = p1 * r + _f(0.368742466)
    p1 = p1 * r + _f(-0.582473278)
    p1 = p1 * r + _f(0.621000469)
    p1 = p1 * r + _f(-0.494451523)
    p1 = p1 * r + _f(0.340488)
    p1 = p1 * r + _f(-0.274112701)
    p1 = p1 * r + _f(0.563825965)
    p2 = r * _f(-10.477664)
    p2 = p2 + _f(12.9772)
    p2 = p2 * r + _f(-7.49551868)
    p2 = p2 * r + _f(2.92101908)
    p2 = p2 * r + _f(-1.01526523)
    p2 = p2 * r + _f(0.42184633)
    p2 = p2 * r + _f(-0.282076746)
    p2 = p2 * r + _f(0.564189494)
    poly = jnp.where(ax < _f(2.0), p1, p2)
    y = zq * poly
    y = jnp.where(underflow, _f(0.0), y)
    y = jnp.where(x < _f(0.0), _f(2.0) - y, y)
    return jnp.where(lt1, erf_branch, y)


def _xla_gelu(x):
    return (_f(0.5) * x) * _xla_erfc((-x) * _f(0.7071067690849304))


def _score_body(x_ref, w1_ref, b1_ref, w2_ref, b2_ref, out_ref):
    HB = BLK // 2
    w1 = w1_ref[...]
    w2 = w2_ref[...]
    for t in range(2):
        xb = x_ref[pl.ds(t * HB, HB), :]
        h = jnp.dot(xb, w1, preferred_element_type=jnp.float32)
        h = h + b1_ref[0:1, :]
        g = _xla_gelu(h)
        s = jnp.dot(g, w2, preferred_element_type=jnp.float32)
        s = s[:, 0] + b2_ref[0, 0]
        bits = lax.bitcast_convert_type(s, jnp.uint32)
        key = jnp.where(s >= 0.0, bits | jnp.uint32(0x80000000), ~bits)
        out_ref[0, 0, pl.ds(t * HB, HB)] = key


def _select_body(keys_ref, t_ref, need_ref):
    ku = keys_ref[...]
    p = jnp.zeros((B, 1), jnp.uint32)
    for j in range(31, -1, -1):
        cand = p | (jnp.uint32(1) << j)
        cnt = jnp.sum((ku >= cand).astype(jnp.int32), axis=1, keepdims=True)
        p = jnp.where(cnt >= K, cand, p)
    cnt_gt = jnp.sum((ku > p).astype(jnp.int32), axis=1, keepdims=True)
    need = (K - cnt_gt).astype(jnp.int32)
    pad = jnp.zeros((8 - B, 1), p.dtype)
    t_ref[...] = jnp.broadcast_to(jnp.concatenate([p, pad], axis=0), (8, 128))
    padn = jnp.zeros((8 - B, 1), need.dtype)
    need_ref[...] = jnp.broadcast_to(
        jnp.concatenate([need, padn], axis=0), (8, 128))


def _sc_body(x_hbm, keys_hbm, t_hbm, need_hbm, out_hbm, idx_hbm,
             keys_v, t_v, need_v, idxs_v, my_v, rows_v,
             gsem0, gsem1, wsem0, wsem1):
    cid = lax.axis_index("c")
    sid = lax.axis_index("s")
    wid = cid * NS + sid
    b = wid // TPB

    @pl.when(wid % TPB == 0)
    def _select():
        pltpu.sync_copy(keys_hbm.at[pl.ds(b * S, S)], keys_v)
        pltpu.sync_copy(t_hbm.at[pl.ds(b * 128, L)], t_v)
        pltpu.sync_copy(need_hbm.at[pl.ds(b * 128, L)], need_v)
        tv = t_v[...]
        nv = need_v[...]
        base = jnp.int32(b * S)

        def step(i, carry):
            off, eq_seen = carry
            kv = keys_v[pl.ds(i * L, L)]
            gt = kv > tv
            eq = kv == tv
            eqi = jnp.where(eq, jnp.int32(1), jnp.int32(0))
            ecs = plsc.cumsum(eqi)
            keep_eq = jnp.logical_and(eq, (ecs + eq_seen) <= nv)
            m = jnp.logical_or(gt, keep_eq)
            mi = jnp.where(m, jnp.int32(1), jnp.int32(0))
            mcs = plsc.cumsum(mi)
            pos = off + (mcs - mi)
            idxv = lax.iota(jnp.int32, L) + (i * L + base)
            plsc.store_scatter(idxs_v, [pos], idxv, mask=m)
            return off + jnp.max(mcs), eq_seen + jnp.max(ecs)

        lax.fori_loop(0, S // L, step, (jnp.int32(0), jnp.int32(0)))
        pltpu.sync_copy(idxs_v.at[pl.ds(0, K)], idx_hbm.at[pl.ds(b * K, K)])

    plsc.subcore_barrier()

    lo = (wid % TPB) * ROWS_PER_TILE
    pltpu.sync_copy(idx_hbm.at[pl.ds(b * K + lo, ROWS_PER_TILE)], my_v)
    nch = ROWS_PER_TILE // GCH
    gsems = (gsem0, gsem1)
    wsems = (wsem0, wsem1)
    bufs = (rows_v.at[0], rows_v.at[1])
    for j in range(2):
        pltpu.async_copy(
            x_hbm.at[my_v.at[pl.ds(j * GCH, GCH)]], bufs[j], gsems[j])
    for j in range(nch):
        bi = j % 2
        pltpu.make_async_copy(
            x_hbm.at[my_v.at[pl.ds(j * GCH, GCH)]], bufs[bi], gsems[bi]).wait()
        dst = out_hbm.at[pl.ds(b * K + lo + j * GCH, GCH), :]
        w = pltpu.async_copy(bufs[bi], dst, wsems[bi])
        if j + 2 < nch:
            w.wait()
            pltpu.async_copy(
                x_hbm.at[my_v.at[pl.ds((j + 2) * GCH, GCH)]], bufs[bi],
                gsems[bi])
        else:
            w.wait()


def _compact_gather(x2d, keys_flat, t_flat, need_flat):
    mesh = plsc.VectorSubcoreMesh(core_axis_name="c", subcore_axis_name="s")
    kern = pl.kernel(
        _sc_body,
        out_type=(
            jax.ShapeDtypeStruct((B * K, H), jnp.float32),
            jax.ShapeDtypeStruct((B * K,), jnp.int32),
        ),
        mesh=mesh,
        compiler_params=pltpu.CompilerParams(needs_layout_passes=False),
        scratch_types=[
            pltpu.VMEM((S,), jnp.uint32),
            pltpu.VMEM((L,), jnp.uint32),
            pltpu.VMEM((L,), jnp.int32),
            pltpu.VMEM((K + L,), jnp.int32),
            pltpu.VMEM((ROWS_PER_TILE,), jnp.int32),
            pltpu.VMEM((2, GCH, H), jnp.float32),
            pltpu.SemaphoreType.DMA,
            pltpu.SemaphoreType.DMA,
            pltpu.SemaphoreType.DMA,
            pltpu.SemaphoreType.DMA,
        ],
    )
    return kern(x2d, keys_flat, t_flat, need_flat)


@jax.jit
def kernel(x, W1, b1, W2, b2):
    x2d = x.reshape(B * S, H)
    b1r = jnp.broadcast_to(b1[None, :], (8, HQ))
    w2p = jnp.pad(W2, ((0, 0), (0, 128 - W2.shape[1])))
    b2r = jnp.broadcast_to(b2[None, :], (8, 128))

    keys3 = pl.pallas_call(
        _score_body,
        grid=(NBLK,),
        in_specs=[
            pl.BlockSpec((BLK, H), lambda i: (i, 0)),
            pl.BlockSpec((H, HQ), lambda i: (0, 0)),
            pl.BlockSpec((8, HQ), lambda i: (0, 0)),
            pl.BlockSpec((HQ, 128), lambda i: (0, 0)),
            pl.BlockSpec((8, 128), lambda i: (0, 0)),
        ],
        out_specs=pl.BlockSpec((1, 1, BLK), lambda i: (i, 0, 0)),
        out_shape=jax.ShapeDtypeStruct((NBLK, 1, BLK), jnp.uint32),
    )(x2d, W1, b1r, w2p, b2r)
    keys = keys3.reshape(B, S)

    t8, need8 = pl.pallas_call(
        _select_body,
        out_shape=(
            jax.ShapeDtypeStruct((8, 128), jnp.uint32),
            jax.ShapeDtypeStruct((8, 128), jnp.int32),
        ),
    )(keys)

    out_flat, _ = _compact_gather(
        x2d, keys.reshape(-1), t8.reshape(-1), need8.reshape(-1))
    return out_flat.reshape(B, K, H)


# speedup vs baseline: 1.0365x; 1.0155x over previous
import functools

import jax
import jax.numpy as jnp
from jax import lax
from jax.experimental import pallas as pl
from jax.experimental.pallas import tpu as pltpu
from jax.experimental.pallas import tpu_sc as plsc

B, S, H = 4, 8192, 768
K = S // 2
HQ = H // 4
BLK = 512
NBLK = (B * S) // BLK

NC, NS, L = 2, 16, 16
TPB = (NC * NS) // B
ROWS_PER_TILE = K // TPB
GCH = 64


def _f(v):
    return jnp.float32(v)


def _xla_erfc(x):
    one = _f(1.0)
    ax = jnp.abs(x)
    lt1 = ax < one
    x2 = x * x
    p = x2 * _f(7.85386146e-05)
    p = p + _f(-0.000801019371)
    p = p * x2 + _f(0.00518832775)
    p = p * x2 + _f(-0.0268538129)
    p = p * x2 + _f(0.112835854)
    p = p * x2 + _f(-0.37612626)
    p = p * x2 + _f(1.12837911)
    erf_branch = one - x * p

    nx2 = -x2
    underflow = nx2 < _f(-88.7228394)
    z = jnp.exp(nx2)
    zq = z * (one / ax)
    r = one / x2
    p1 = r * _f(0.0232682)
    p1 = p1 + _f(-0.138703942)
    p1 = p1 * r + _f(0.368742466)
    p1 = p1 * r + _f(-0.582473278)
    p1 = p1 * r + _f(0.621000469)
    p1 = p1 * r + _f(-0.494451523)
    p1 = p1 * r + _f(0.340488)
    p1 = p1 * r + _f(-0.274112701)
    p1 = p1 * r + _f(0.563825965)
    p2 = r * _f(-10.477664)
    p2 = p2 + _f(12.9772)
    p2 = p2 * r + _f(-7.49551868)
    p2 = p2 * r + _f(2.92101908)
    p2 = p2 * r + _f(-1.01526523)
    p2 = p2 * r + _f(0.42184633)
    p2 = p2 * r + _f(-0.282076746)
    p2 = p2 * r + _f(0.564189494)
    poly = jnp.where(ax < _f(2.0), p1, p2)
    y = zq * poly
    y = jnp.where(underflow, _f(0.0), y)
    y = jnp.where(x < _f(0.0), _f(2.0) - y, y)
    return jnp.where(lt1, erf_branch, y)


def _xla_gelu(x):
    return (_f(0.5) * x) * _xla_erfc((-x) * _f(0.7071067690849304))


def _score_body(x_ref, w1_ref, b1_ref, w2_ref, b2_ref,
                out_ref, t_ref, need_ref, keys_scr):
    i = pl.program_id(0)
    HB = BLK // 2
    w1 = w1_ref[...]
    w2 = w2_ref[...]
    for t in range(2):
        xb = x_ref[pl.ds(t * HB, HB), :]
        h = jnp.dot(xb, w1, preferred_element_type=jnp.float32)
        h = h + b1_ref[0:1, :]
        g = _xla_gelu(h)
        s = jnp.dot(g, w2, preferred_element_type=jnp.float32)
        s = s[:, 0] + b2_ref[0, 0]
        bits = lax.bitcast_convert_type(s, jnp.uint32)
        key = jnp.where(s >= 0.0, bits | jnp.uint32(0x80000000), ~bits)
        out_ref[0, 0, pl.ds(t * HB, HB)] = key
        keys_scr[pl.ds(i, 1), pl.ds(t * HB, HB)] = key.reshape(1, HB)

    @pl.when(i == NBLK - 1)
    def _select():
        ku = keys_scr[...].reshape(B, S)
        p = jnp.zeros((B, 1), jnp.uint32)
        for j in range(31, -1, -1):
            cand = p | (jnp.uint32(1) << j)
            cnt = jnp.sum((ku >= cand).astype(jnp.int32), axis=1,
                          keepdims=True)
            p = jnp.where(cnt >= K, cand, p)
        cnt_gt = jnp.sum((ku > p).astype(jnp.int32), axis=1, keepdims=True)
        need = (K - cnt_gt).astype(jnp.int32)
        pad = jnp.zeros((8 - B, 1), p.dtype)
        t_ref[...] = jnp.broadcast_to(
            jnp.concatenate([p, pad], axis=0), (8, 128))
        padn = jnp.zeros((8 - B, 1), need.dtype)
        need_ref[...] = jnp.broadcast_to(
            jnp.concatenate([need, padn], axis=0), (8, 128))


def _select_body(keys_ref, t_ref, need_ref):
    ku = keys_ref[...]
    p = jnp.zeros((B, 1), jnp.uint32)
    for j in range(31, -1, -1):
        cand = p | (jnp.uint32(1) << j)
        cnt = jnp.sum((ku >= cand).astype(jnp.int32), axis=1, keepdims=True)
        p = jnp.where(cnt >= K, cand, p)
    cnt_gt = jnp.sum((ku > p).astype(jnp.int32), axis=1, keepdims=True)
    need = (K - cnt_gt).astype(jnp.int32)
    pad = jnp.zeros((8 - B, 1), p.dtype)
    t_ref[...] = jnp.broadcast_to(jnp.concatenate([p, pad], axis=0), (8, 128))
    padn = jnp.zeros((8 - B, 1), need.dtype)
    need_ref[...] = jnp.broadcast_to(
        jnp.concatenate([need, padn], axis=0), (8, 128))


def _sc_body(x_hbm, keys_hbm, t_hbm, need_hbm, out_hbm, idx_hbm,
             keys_v, t_v, need_v, idxs_v, my_v, rows_v,
             gsem0, gsem1, wsem0, wsem1):
    cid = lax.axis_index("c")
    sid = lax.axis_index("s")
    wid = cid * NS + sid
    b = wid // TPB

    @pl.when(wid % TPB == 0)
    def _select():
        pltpu.sync_copy(keys_hbm.at[pl.ds(b * S, S)], keys_v)
        pltpu.sync_copy(t_hbm.at[pl.ds(b * 128, L)], t_v)
        pltpu.sync_copy(need_hbm.at[pl.ds(b * 128, L)], need_v)
        tv = t_v[...]
        nv = need_v[...]
        base = jnp.int32(b * S)

        def step(i, carry):
            off, eq_seen = carry
            kv = keys_v[pl.ds(i * L, L)]
            gt = kv > tv
            eq = kv == tv
            eqi = jnp.where(eq, jnp.int32(1), jnp.int32(0))
            ecs = plsc.cumsum(eqi)
            keep_eq = jnp.logical_and(eq, (ecs + eq_seen) <= nv)
            m = jnp.logical_or(gt, keep_eq)
            mi = jnp.where(m, jnp.int32(1), jnp.int32(0))
            mcs = plsc.cumsum(mi)
            pos = off + (mcs - mi)
            idxv = lax.iota(jnp.int32, L) + (i * L + base)
            plsc.store_scatter(idxs_v, [pos], idxv, mask=m)
            return off + jnp.max(mcs), eq_seen + jnp.max(ecs)

        lax.fori_loop(0, S // L, step, (jnp.int32(0), jnp.int32(0)))
        pltpu.sync_copy(idxs_v.at[pl.ds(0, K)], idx_hbm.at[pl.ds(b * K, K)])

    plsc.subcore_barrier()

    lo = (wid % TPB) * ROWS_PER_TILE
    pltpu.sync_copy(idx_hbm.at[pl.ds(b * K + lo, ROWS_PER_TILE)], my_v)
    nch = ROWS_PER_TILE // GCH
    gsems = (gsem0, gsem1)
    wsems = (wsem0, wsem1)
    bufs = (rows_v.at[0], rows_v.at[1])
    for j in range(2):
        pltpu.async_copy(
            x_hbm.at[my_v.at[pl.ds(j * GCH, GCH)]], bufs[j], gsems[j])
    for j in range(nch):
        bi = j % 2
        pltpu.make_async_copy(
            x_hbm.at[my_v.at[pl.ds(j * GCH, GCH)]], bufs[bi], gsems[bi]).wait()
        dst = out_hbm.at[pl.ds(b * K + lo + j * GCH, GCH), :]
        w = pltpu.async_copy(bufs[bi], dst, wsems[bi])
        if j + 2 < nch:
            w.wait()
            pltpu.async_copy(
                x_hbm.at[my_v.at[pl.ds((j + 2) * GCH, GCH)]], bufs[bi],
                gsems[bi])
        else:
            w.wait()


def _compact_gather(x2d, keys_flat, t_flat, need_flat):
    mesh = plsc.VectorSubcoreMesh(core_axis_name="c", subcore_axis_name="s")
    kern = pl.kernel(
        _sc_body,
        out_type=(
            jax.ShapeDtypeStruct((B * K, H), jnp.float32),
            jax.ShapeDtypeStruct((B * K,), jnp.int32),
        ),
        mesh=mesh,
        compiler_params=pltpu.CompilerParams(needs_layout_passes=False),
        scratch_types=[
            pltpu.VMEM((S,), jnp.uint32),
            pltpu.VMEM((L,), jnp.uint32),
            pltpu.VMEM((L,), jnp.int32),
            pltpu.VMEM((K + L,), jnp.int32),
            pltpu.VMEM((ROWS_PER_TILE,), jnp.int32),
            pltpu.VMEM((2, GCH, H), jnp.float32),
            pltpu.SemaphoreType.DMA,
            pltpu.SemaphoreType.DMA,
            pltpu.SemaphoreType.DMA,
            pltpu.SemaphoreType.DMA,
        ],
    )
    return kern(x2d, keys_flat, t_flat, need_flat)


@jax.jit
def kernel(x, W1, b1, W2, b2):
    x2d = x.reshape(B * S, H)
    b1r = jnp.broadcast_to(b1[None, :], (8, HQ))
    w2p = jnp.pad(W2, ((0, 0), (0, 128 - W2.shape[1])))
    b2r = jnp.broadcast_to(b2[None, :], (8, 128))

    keys3, t8, need8 = pl.pallas_call(
        _score_body,
        grid=(NBLK,),
        in_specs=[
            pl.BlockSpec((BLK, H), lambda i: (i, 0)),
            pl.BlockSpec((H, HQ), lambda i: (0, 0)),
            pl.BlockSpec((8, HQ), lambda i: (0, 0)),
            pl.BlockSpec((HQ, 128), lambda i: (0, 0)),
            pl.BlockSpec((8, 128), lambda i: (0, 0)),
        ],
        out_specs=(
            pl.BlockSpec((1, 1, BLK), lambda i: (i, 0, 0)),
            pl.BlockSpec((8, 128), lambda i: (0, 0)),
            pl.BlockSpec((8, 128), lambda i: (0, 0)),
        ),
        out_shape=(
            jax.ShapeDtypeStruct((NBLK, 1, BLK), jnp.uint32),
            jax.ShapeDtypeStruct((8, 128), jnp.uint32),
            jax.ShapeDtypeStruct((8, 128), jnp.int32),
        ),
        scratch_shapes=[pltpu.VMEM((NBLK, BLK), jnp.uint32)],
    )(x2d, W1, b1r, w2p, b2r)
    keys = keys3.reshape(B, S)

    out_flat, _ = _compact_gather(
        x2d, keys.reshape(-1), t8.reshape(-1), need8.reshape(-1))
    return out_flat.reshape(B, K, H)


# speedup vs baseline: 1.0607x; 1.0234x over previous
import functools

import jax
import jax.numpy as jnp
from jax import lax
from jax.experimental import pallas as pl
from jax.experimental.pallas import tpu as pltpu
from jax.experimental.pallas import tpu_sc as plsc

B, S, H = 4, 8192, 768
K = S // 2
HQ = H // 4
BLK = 512
NBLK = (B * S) // BLK

NC, NS, L = 2, 16, 16
TPB = (NC * NS) // B
ROWS_PER_TILE = K // TPB
GCH = 64


def _f(v):
    return jnp.float32(v)


def _xla_erfc(x):
    one = _f(1.0)
    ax = jnp.abs(x)
    lt1 = ax < one
    x2 = x * x
    p = x2 * _f(7.85386146e-05)
    p = p + _f(-0.000801019371)
    p = p * x2 + _f(0.00518832775)
    p = p * x2 + _f(-0.0268538129)
    p = p * x2 + _f(0.112835854)
    p = p * x2 + _f(-0.37612626)
    p = p * x2 + _f(1.12837911)
    erf_branch = one - x * p

    nx2 = -x2
    underflow = nx2 < _f(-88.7228394)
    z = jnp.exp(nx2)
    zq = z * (one / ax)
    r = one / x2
    p1 = r * _f(0.0232682)
    p1 = p1 + _f(-0.138703942)
    p1 = p1 * r + _f(0.368742466)
    p1 = p1 * r + _f(-0.582473278)
    p1 = p1 * r + _f(0.621000469)
    p1 = p1 * r + _f(-0.494451523)
    p1 = p1 * r + _f(0.340488)
    p1 = p1 * r + _f(-0.274112701)
    p1 = p1 * r + _f(0.563825965)
    p2 = r * _f(-10.477664)
    p2 = p2 + _f(12.9772)
    p2 = p2 * r + _f(-7.49551868)
    p2 = p2 * r + _f(2.92101908)
    p2 = p2 * r + _f(-1.01526523)
    p2 = p2 * r + _f(0.42184633)
    p2 = p2 * r + _f(-0.282076746)
    p2 = p2 * r + _f(0.564189494)
    poly = jnp.where(ax < _f(2.0), p1, p2)
    y = zq * poly
    y = jnp.where(underflow, _f(0.0), y)
    y = jnp.where(x < _f(0.0), _f(2.0) - y, y)
    return jnp.where(lt1, erf_branch, y)


def _xla_gelu(x):
    return (_f(0.5) * x) * _xla_erfc((-x) * _f(0.7071067690849304))


def _score_body(x_ref, w1_ref, b1_ref, w2_ref, b2_ref,
                out_ref, t_ref, need_ref, keys_scr):
    i = pl.program_id(0)
    HB = BLK // 2
    w1 = w1_ref[...]
    w2 = w2_ref[...]
    for t in range(2):
        xb = x_ref[pl.ds(t * HB, HB), :]
        h = jnp.dot(xb, w1, preferred_element_type=jnp.float32)
        h = h + b1_ref[0:1, :]
        g = _xla_gelu(h)
        s = jnp.dot(g, w2, preferred_element_type=jnp.float32)
        s = s[:, 0] + b2_ref[0, 0]
        bits = lax.bitcast_convert_type(s, jnp.uint32)
        key = jnp.where(s >= 0.0, bits | jnp.uint32(0x80000000), ~bits)
        out_ref[0, 0, pl.ds(t * HB, HB)] = key
        keys_scr[pl.ds(i, 1), pl.ds(t * HB, HB)] = key.reshape(1, HB)

    @pl.when(i == NBLK - 1)
    def _select():
        ku = keys_scr[...].reshape(B, S)
        p = jnp.zeros((B, 1), jnp.uint32)
        for j in range(31, -1, -1):
            cand = p | (jnp.uint32(1) << j)
            cnt = jnp.sum((ku >= cand).astype(jnp.int32), axis=1,
                          keepdims=True)
            p = jnp.where(cnt >= K, cand, p)
        cnt_gt = jnp.sum((ku > p).astype(jnp.int32), axis=1, keepdims=True)
        need = (K - cnt_gt).astype(jnp.int32)
        pad = jnp.zeros((8 - B, 1), p.dtype)
        t_ref[...] = jnp.broadcast_to(
            jnp.concatenate([p, pad], axis=0), (8, 128))
        padn = jnp.zeros((8 - B, 1), need.dtype)
        need_ref[...] = jnp.broadcast_to(
            jnp.concatenate([need, padn], axis=0), (8, 128))


def _select_body(keys_ref, t_ref, need_ref):
    ku = keys_ref[...]
    p = jnp.zeros((B, 1), jnp.uint32)
    for j in range(31, -1, -1):
        cand = p | (jnp.uint32(1) << j)
        cnt = jnp.sum((ku >= cand).astype(jnp.int32), axis=1, keepdims=True)
        p = jnp.where(cnt >= K, cand, p)
    cnt_gt = jnp.sum((ku > p).astype(jnp.int32), axis=1, keepdims=True)
    need = (K - cnt_gt).astype(jnp.int32)
    pad = jnp.zeros((8 - B, 1), p.dtype)
    t_ref[...] = jnp.broadcast_to(jnp.concatenate([p, pad], axis=0), (8, 128))
    padn = jnp.zeros((8 - B, 1), need.dtype)
    need_ref[...] = jnp.broadcast_to(
        jnp.concatenate([need, padn], axis=0), (8, 128))


CH = S // TPB


def _sc_body(x_hbm, keys_hbm, t_hbm, need_hbm, out_hbm,
             keys_v, t_v, need_v, idxs_l, cnt_st, cnt_v, staged_v, my_v,
             rows_v, shr_cnt, shr_idx, gsem0, gsem1, wsem0, wsem1):
    cid = lax.axis_index("c")
    sid = lax.axis_index("s")
    wid = cid * NS + sid
    b = wid // TPB
    c0 = (wid % TPB) * CH
    lanes = lax.iota(jnp.int32, L)
    one = jnp.int32(1)
    zero = jnp.int32(0)

    pltpu.sync_copy(keys_hbm.at[pl.ds(b * S + c0, CH)], keys_v)
    pltpu.sync_copy(t_hbm.at[pl.ds(b * 128, L)], t_v)
    pltpu.sync_copy(need_hbm.at[pl.ds(b * 128, L)], need_v)
    tv = t_v[...]
    nv = need_v[...]

    def stepA(i, carry):
        ag, ae = carry
        kv = keys_v[pl.ds(i * L, L)]
        return (ag + jnp.where(kv > tv, one, zero),
                ae + jnp.where(kv == tv, one, zero))

    zv = jnp.zeros((L,), jnp.int32)
    ag, ae = lax.fori_loop(0, CH // L, stepA, (zv, zv))
    gt_cnt = jnp.sum(ag)
    eq_cnt = jnp.sum(ae)
    cnt_st[...] = jnp.where(lanes == 0, gt_cnt,
                            jnp.where(lanes == 1, eq_cnt, zero))
    pltpu.sync_copy(cnt_st, shr_cnt.at[sid])
    plsc.subcore_barrier()

    pltpu.sync_copy(shr_cnt, cnt_v)
    gtv = plsc.load_gather(cnt_v, [lanes, jnp.zeros((L,), jnp.int32)])
    eqv = plsc.load_gather(cnt_v, [lanes, jnp.ones((L,), jnp.int32)])
    g0 = (sid // TPB) * TPB
    in_group = jnp.logical_and(lanes >= g0, lanes < g0 + TPB)
    eqv_g = jnp.where(in_group, eqv, zero)
    ecs_g = plsc.cumsum(eqv_g)
    eq_before_v = ecs_g - eqv_g
    eqkeep = jnp.minimum(jnp.maximum(nv - eq_before_v, zero), eqv)
    kept_v = jnp.where(in_group, gtv + eqkeep, zero)
    pcs = plsc.cumsum(kept_v)
    p_excl = pcs - kept_v
    my_eq_before = jnp.sum(jnp.where(lanes == sid, eq_before_v, zero))

    base = jnp.int32(b * S) + c0

    def stepB(i, carry):
        off, eq_seen = carry
        kv = keys_v[pl.ds(i * L, L)]
        gt = kv > tv
        eq = kv == tv
        eqi = jnp.where(eq, one, zero)
        ecs = plsc.cumsum(eqi)
        keep_eq = jnp.logical_and(eq, (ecs + eq_seen) <= nv)
        m = jnp.logical_or(gt, keep_eq)
        mi = jnp.where(m, one, zero)
        mcs = plsc.cumsum(mi)
        pos = off + (mcs - mi)
        idxv = lanes + (i * L + base)
        plsc.store_scatter(idxs_l, [pos], idxv, mask=m)
        return off + jnp.max(mcs), eq_seen + jnp.max(ecs)

    lax.fori_loop(0, CH // L, stepB, (zero, my_eq_before))
    pltpu.sync_copy(idxs_l.at[pl.ds(0, CH)], shr_idx.at[sid])
    plsc.subcore_barrier()

    lo = (wid % TPB) * ROWS_PER_TILE
    pltpu.sync_copy(shr_idx.at[pl.ds(g0, TPB)], staged_v)
    p_scal = [jnp.sum(jnp.where(lanes == g0 + r, p_excl, zero))
              for r in range(TPB)]
    for j in range(ROWS_PER_TILE // L):
        rankv = lanes + (lo + j * L)
        srcv = jnp.full((L,), -1, jnp.int32)
        basev = jnp.zeros((L,), jnp.int32)
        for r in range(TPB):
            hit = rankv >= p_scal[r]
            srcv = srcv + jnp.where(hit, one, zero)
            basev = jnp.maximum(basev, jnp.where(hit, p_scal[r], zero))
        vals = plsc.load_gather(staged_v, [srcv, rankv - basev])
        my_v[pl.ds(j * L, L)] = vals
    nch = ROWS_PER_TILE // GCH
    gsems = (gsem0, gsem1)
    wsems = (wsem0, wsem1)
    bufs = (rows_v.at[0], rows_v.at[1])
    for j in range(2):
        pltpu.async_copy(
            x_hbm.at[my_v.at[pl.ds(j * GCH, GCH)]], bufs[j], gsems[j])
    for j in range(nch):
        bi = j % 2
        pltpu.make_async_copy(
            x_hbm.at[my_v.at[pl.ds(j * GCH, GCH)]], bufs[bi], gsems[bi]).wait()
        dst = out_hbm.at[pl.ds(b * K + lo + j * GCH, GCH), :]
        w = pltpu.async_copy(bufs[bi], dst, wsems[bi])
        if j + 2 < nch:
            w.wait()
            pltpu.async_copy(
                x_hbm.at[my_v.at[pl.ds((j + 2) * GCH, GCH)]], bufs[bi],
                gsems[bi])
        else:
            w.wait()


def _compact_gather(x2d, keys_flat, t_flat, need_flat):
    mesh = plsc.VectorSubcoreMesh(core_axis_name="c", subcore_axis_name="s")
    kern = pl.kernel(
        _sc_body,
        out_type=jax.ShapeDtypeStruct((B * K, H), jnp.float32),
        mesh=mesh,
        compiler_params=pltpu.CompilerParams(needs_layout_passes=False),
        scratch_types=[
            pltpu.VMEM((CH,), jnp.uint32),
            pltpu.VMEM((L,), jnp.uint32),
            pltpu.VMEM((L,), jnp.int32),
            pltpu.VMEM((CH + L,), jnp.int32),
            pltpu.VMEM((L,), jnp.int32),
            pltpu.VMEM((NS, L), jnp.int32),
            pltpu.VMEM((TPB, CH), jnp.int32),
            pltpu.VMEM((ROWS_PER_TILE,), jnp.int32),
            pltpu.VMEM((2, GCH, H), jnp.float32),
            pltpu.VMEM_SHARED((NS, L), jnp.int32),
            pltpu.VMEM_SHARED((NS, CH), jnp.int32),
            pltpu.SemaphoreType.DMA,
            pltpu.SemaphoreType.DMA,
            pltpu.SemaphoreType.DMA,
            pltpu.SemaphoreType.DMA,
        ],
    )
    return kern(x2d, keys_flat, t_flat, need_flat)


@jax.jit
def kernel(x, W1, b1, W2, b2):
    x2d = x.reshape(B * S, H)
    b1r = jnp.broadcast_to(b1[None, :], (8, HQ))
    w2p = jnp.pad(W2, ((0, 0), (0, 128 - W2.shape[1])))
    b2r = jnp.broadcast_to(b2[None, :], (8, 128))

    keys3, t8, need8 = pl.pallas_call(
        _score_body,
        grid=(NBLK,),
        in_specs=[
            pl.BlockSpec((BLK, H), lambda i: (i, 0)),
            pl.BlockSpec((H, HQ), lambda i: (0, 0)),
            pl.BlockSpec((8, HQ), lambda i: (0, 0)),
            pl.BlockSpec((HQ, 128), lambda i: (0, 0)),
            pl.BlockSpec((8, 128), lambda i: (0, 0)),
        ],
        out_specs=(
            pl.BlockSpec((1, 1, BLK), lambda i: (i, 0, 0)),
            pl.BlockSpec((8, 128), lambda i: (0, 0)),
            pl.BlockSpec((8, 128), lambda i: (0, 0)),
        ),
        out_shape=(
            jax.ShapeDtypeStruct((NBLK, 1, BLK), jnp.uint32),
            jax.ShapeDtypeStruct((8, 128), jnp.uint32),
            jax.ShapeDtypeStruct((8, 128), jnp.int32),
        ),
        scratch_shapes=[pltpu.VMEM((NBLK, BLK), jnp.uint32)],
    )(x2d, W1, b1r, w2p, b2r)
    keys = keys3.reshape(B, S)

    out_flat = _compact_gather(
        x2d, keys.reshape(-1), t8.reshape(-1), need8.reshape(-1))
    return out_flat.reshape(B, K, H)


# speedup vs baseline: 1.1566x; 1.0904x over previous
import functools

import jax
import jax.numpy as jnp
from jax import lax
from jax.experimental import pallas as pl
from jax.experimental.pallas import tpu as pltpu
from jax.experimental.pallas import tpu_sc as plsc

B, S, H = 4, 8192, 768
K = S // 2
HQ = H // 4
BLK = 1024
NBLK = (B * S) // BLK

NC, NS, L = 2, 16, 16
TPB = (NC * NS) // B
ROWS_PER_TILE = K // TPB
GCH = 64


def _f(v):
    return jnp.float32(v)


def _xla_erfc(x):
    one = _f(1.0)
    ax = jnp.abs(x)
    lt1 = ax < one
    x2 = x * x
    p = x2 * _f(7.85386146e-05)
    p = p + _f(-0.000801019371)
    p = p * x2 + _f(0.00518832775)
    p = p * x2 + _f(-0.0268538129)
    p = p * x2 + _f(0.112835854)
    p = p * x2 + _f(-0.37612626)
    p = p * x2 + _f(1.12837911)
    erf_branch = one - x * p

    nx2 = -x2
    underflow = nx2 < _f(-88.7228394)
    z = jnp.exp(nx2)
    zq = z * (one / ax)
    r = one / x2
    p1 = r * _f(0.0232682)
    p1 = p1 + _f(-0.138703942)
    p1 = p1 * r + _f(0.368742466)
    p1 = p1 * r + _f(-0.582473278)
    p1 = p1 * r + _f(0.621000469)
    p1 = p1 * r + _f(-0.494451523)
    p1 = p1 * r + _f(0.340488)
    p1 = p1 * r + _f(-0.274112701)
    p1 = p1 * r + _f(0.563825965)
    p2 = r * _f(-10.477664)
    p2 = p2 + _f(12.9772)
    p2 = p2 * r + _f(-7.49551868)
    p2 = p2 * r + _f(2.92101908)
    p2 = p2 * r + _f(-1.01526523)
    p2 = p2 * r + _f(0.42184633)
    p2 = p2 * r + _f(-0.282076746)
    p2 = p2 * r + _f(0.564189494)
    poly = jnp.where(ax < _f(2.0), p1, p2)
    y = zq * poly
    y = jnp.where(underflow, _f(0.0), y)
    y = jnp.where(x < _f(0.0), _f(2.0) - y, y)
    return jnp.where(lt1, erf_branch, y)


def _xla_gelu(x):
    return (_f(0.5) * x) * _xla_erfc((-x) * _f(0.7071067690849304))


def _score_body(x_ref, w1_ref, b1_ref, w2_ref, b2_ref,
                out_ref, t_ref, need_ref, keys_scr):
    i = pl.program_id(0)
    HB = BLK // 2
    w1 = w1_ref[...]
    w2 = w2_ref[...]
    for t in range(2):
        xb = x_ref[pl.ds(t * HB, HB), :]
        h = jnp.dot(xb, w1, preferred_element_type=jnp.float32)
        h = h + b1_ref[0:1, :]
        g = _xla_gelu(h)
        s = jnp.dot(g, w2, preferred_element_type=jnp.float32)
        s = s[:, 0] + b2_ref[0, 0]
        bits = lax.bitcast_convert_type(s, jnp.uint32)
        key = jnp.where(s >= 0.0, bits | jnp.uint32(0x80000000), ~bits)
        out_ref[0, 0, pl.ds(t * HB, HB)] = key
        keys_scr[pl.ds(i, 1), pl.ds(t * HB, HB)] = key.reshape(1, HB)

    @pl.when(i == NBLK - 1)
    def _select():
        ku = keys_scr[...].reshape(B, S)
        p = jnp.zeros((B, 1), jnp.uint32)
        for j in range(31, -1, -1):
            cand = p | (jnp.uint32(1) << j)
            cnt = jnp.sum((ku >= cand).astype(jnp.int32), axis=1,
                          keepdims=True)
            p = jnp.where(cnt >= K, cand, p)
        cnt_gt = jnp.sum((ku > p).astype(jnp.int32), axis=1, keepdims=True)
        need = (K - cnt_gt).astype(jnp.int32)
        pad = jnp.zeros((8 - B, 1), p.dtype)
        t_ref[...] = jnp.broadcast_to(
            jnp.concatenate([p, pad], axis=0), (8, 128))
        padn = jnp.zeros((8 - B, 1), need.dtype)
        need_ref[...] = jnp.broadcast_to(
            jnp.concatenate([need, padn], axis=0), (8, 128))


def _select_body(keys_ref, t_ref, need_ref):
    ku = keys_ref[...]
    p = jnp.zeros((B, 1), jnp.uint32)
    for j in range(31, -1, -1):
        cand = p | (jnp.uint32(1) << j)
        cnt = jnp.sum((ku >= cand).astype(jnp.int32), axis=1, keepdims=True)
        p = jnp.where(cnt >= K, cand, p)
    cnt_gt = jnp.sum((ku > p).astype(jnp.int32), axis=1, keepdims=True)
    need = (K - cnt_gt).astype(jnp.int32)
    pad = jnp.zeros((8 - B, 1), p.dtype)
    t_ref[...] = jnp.broadcast_to(jnp.concatenate([p, pad], axis=0), (8, 128))
    padn = jnp.zeros((8 - B, 1), need.dtype)
    need_ref[...] = jnp.broadcast_to(
        jnp.concatenate([need, padn], axis=0), (8, 128))


CH = S // TPB


def _sc_body(x_hbm, keys_hbm, t_hbm, need_hbm, out_hbm,
             keys_v, t_v, need_v, idxs_l, cnt_st, cnt_v, staged_v, my_v,
             rows_v, shr_cnt, shr_idx, gsem0, gsem1, wsem0, wsem1):
    cid = lax.axis_index("c")
    sid = lax.axis_index("s")
    wid = cid * NS + sid
    b = wid // TPB
    c0 = (wid % TPB) * CH
    lanes = lax.iota(jnp.int32, L)
    one = jnp.int32(1)
    zero = jnp.int32(0)

    pltpu.sync_copy(keys_hbm.at[pl.ds(b * S + c0, CH)], keys_v)
    pltpu.sync_copy(t_hbm.at[pl.ds(b * 128, L)], t_v)
    pltpu.sync_copy(need_hbm.at[pl.ds(b * 128, L)], need_v)
    tv = t_v[...]
    nv = need_v[...]

    def stepA(i, carry):
        ag, ae = carry
        kv = keys_v[pl.ds(i * L, L)]
        return (ag + jnp.where(kv > tv, one, zero),
                ae + jnp.where(kv == tv, one, zero))

    zv = jnp.zeros((L,), jnp.int32)
    ag, ae = lax.fori_loop(0, CH // L, stepA, (zv, zv))
    gt_cnt = jnp.sum(ag)
    eq_cnt = jnp.sum(ae)
    cnt_st[...] = jnp.where(lanes == 0, gt_cnt,
                            jnp.where(lanes == 1, eq_cnt, zero))
    pltpu.sync_copy(cnt_st, shr_cnt.at[sid])
    plsc.subcore_barrier()

    pltpu.sync_copy(shr_cnt, cnt_v)
    gtv = plsc.load_gather(cnt_v, [lanes, jnp.zeros((L,), jnp.int32)])
    eqv = plsc.load_gather(cnt_v, [lanes, jnp.ones((L,), jnp.int32)])
    g0 = (sid // TPB) * TPB
    in_group = jnp.logical_and(lanes >= g0, lanes < g0 + TPB)
    eqv_g = jnp.where(in_group, eqv, zero)
    ecs_g = plsc.cumsum(eqv_g)
    eq_before_v = ecs_g - eqv_g
    eqkeep = jnp.minimum(jnp.maximum(nv - eq_before_v, zero), eqv)
    kept_v = jnp.where(in_group, gtv + eqkeep, zero)
    pcs = plsc.cumsum(kept_v)
    p_excl = pcs - kept_v
    my_eq_before = jnp.sum(jnp.where(lanes == sid, eq_before_v, zero))

    base = jnp.int32(b * S) + c0

    def stepB(i, carry):
        off, eq_seen = carry
        kv = keys_v[pl.ds(i * L, L)]
        gt = kv > tv
        eq = kv == tv
        eqi = jnp.where(eq, one, zero)
        ecs = plsc.cumsum(eqi)
        keep_eq = jnp.logical_and(eq, (ecs + eq_seen) <= nv)
        m = jnp.logical_or(gt, keep_eq)
        mi = jnp.where(m, one, zero)
        mcs = plsc.cumsum(mi)
        pos = off + (mcs - mi)
        idxv = lanes + (i * L + base)
        plsc.store_scatter(idxs_l, [pos], idxv, mask=m)
        return off + jnp.max(mcs), eq_seen + jnp.max(ecs)

    lax.fori_loop(0, CH // L, stepB, (zero, my_eq_before))
    pltpu.sync_copy(idxs_l.at[pl.ds(0, CH)], shr_idx.at[sid])
    plsc.subcore_barrier()

    lo = (wid % TPB) * ROWS_PER_TILE
    pltpu.sync_copy(shr_idx.at[pl.ds(g0, TPB)], staged_v)
    p_scal = [jnp.sum(jnp.where(lanes == g0 + r, p_excl, zero))
              for r in range(TPB)]
    for j in range(ROWS_PER_TILE // L):
        rankv = lanes + (lo + j * L)
        srcv = jnp.full((L,), -1, jnp.int32)
        basev = jnp.zeros((L,), jnp.int32)
        for r in range(TPB):
            hit = rankv >= p_scal[r]
            srcv = srcv + jnp.where(hit, one, zero)
            basev = jnp.maximum(basev, jnp.where(hit, p_scal[r], zero))
        vals = plsc.load_gather(staged_v, [srcv, rankv - basev])
        my_v[pl.ds(j * L, L)] = vals
    nch = ROWS_PER_TILE // GCH
    gsems = (gsem0, gsem1)
    wsems = (wsem0, wsem1)
    bufs = (rows_v.at[0], rows_v.at[1])
    for j in range(2):
        pltpu.async_copy(
            x_hbm.at[my_v.at[pl.ds(j * GCH, GCH)]], bufs[j], gsems[j])
    for j in range(nch):
        bi = j % 2
        pltpu.make_async_copy(
            x_hbm.at[my_v.at[pl.ds(j * GCH, GCH)]], bufs[bi], gsems[bi]).wait()
        dst = out_hbm.at[pl.ds(b * K + lo + j * GCH, GCH), :]
        w = pltpu.async_copy(bufs[bi], dst, wsems[bi])
        if j + 2 < nch:
            w.wait()
            pltpu.async_copy(
                x_hbm.at[my_v.at[pl.ds((j + 2) * GCH, GCH)]], bufs[bi],
                gsems[bi])
        else:
            w.wait()


def _compact_gather(x2d, keys_flat, t_flat, need_flat):
    mesh = plsc.VectorSubcoreMesh(core_axis_name="c", subcore_axis_name="s")
    kern = pl.kernel(
        _sc_body,
        out_type=jax.ShapeDtypeStruct((B * K, H), jnp.float32),
        mesh=mesh,
        compiler_params=pltpu.CompilerParams(needs_layout_passes=False),
        scratch_types=[
            pltpu.VMEM((CH,), jnp.uint32),
            pltpu.VMEM((L,), jnp.uint32),
            pltpu.VMEM((L,), jnp.int32),
            pltpu.VMEM((CH + L,), jnp.int32),
            pltpu.VMEM((L,), jnp.int32),
            pltpu.VMEM((NS, L), jnp.int32),
            pltpu.VMEM((TPB, CH), jnp.int32),
            pltpu.VMEM((ROWS_PER_TILE,), jnp.int32),
            pltpu.VMEM((2, GCH, H), jnp.float32),
            pltpu.VMEM_SHARED((NS, L), jnp.int32),
            pltpu.VMEM_SHARED((NS, CH), jnp.int32),
            pltpu.SemaphoreType.DMA,
            pltpu.SemaphoreType.DMA,
            pltpu.SemaphoreType.DMA,
            pltpu.SemaphoreType.DMA,
        ],
    )
    return kern(x2d, keys_flat, t_flat, need_flat)


@jax.jit
def kernel(x, W1, b1, W2, b2):
    x2d = x.reshape(B * S, H)
    b1r = jnp.broadcast_to(b1[None, :], (8, HQ))
    w2p = jnp.pad(W2, ((0, 0), (0, 128 - W2.shape[1])))
    b2r = jnp.broadcast_to(b2[None, :], (8, 128))

    keys3, t8, need8 = pl.pallas_call(
        _score_body,
        grid=(NBLK,),
        in_specs=[
            pl.BlockSpec((BLK, H), lambda i: (i, 0)),
            pl.BlockSpec((H, HQ), lambda i: (0, 0)),
            pl.BlockSpec((8, HQ), lambda i: (0, 0)),
            pl.BlockSpec((HQ, 128), lambda i: (0, 0)),
            pl.BlockSpec((8, 128), lambda i: (0, 0)),
        ],
        out_specs=(
            pl.BlockSpec((1, 1, BLK), lambda i: (i, 0, 0)),
            pl.BlockSpec((8, 128), lambda i: (0, 0)),
            pl.BlockSpec((8, 128), lambda i: (0, 0)),
        ),
        out_shape=(
            jax.ShapeDtypeStruct((NBLK, 1, BLK), jnp.uint32),
            jax.ShapeDtypeStruct((8, 128), jnp.uint32),
            jax.ShapeDtypeStruct((8, 128), jnp.int32),
        ),
        scratch_shapes=[pltpu.VMEM((NBLK, BLK), jnp.uint32)],
    )(x2d, W1, b1r, w2p, b2r)
    keys = keys3.reshape(B, S)

    out_flat = _compact_gather(
        x2d, keys.reshape(-1), t8.reshape(-1), need8.reshape(-1))
    return out_flat.reshape(B, K, H)


# speedup vs baseline: 1.1748x; 1.0157x over previous
import functools

import jax
import jax.numpy as jnp
from jax import lax
from jax.experimental import pallas as pl
from jax.experimental.pallas import tpu as pltpu
from jax.experimental.pallas import tpu_sc as plsc

B, S, H = 4, 8192, 768
K = S // 2
HQ = H // 4
BLK = 2048
NBLK = (B * S) // BLK

NC, NS, L = 2, 16, 16
TPB = (NC * NS) // B
ROWS_PER_TILE = K // TPB
GCH = 64


def _f(v):
    return jnp.float32(v)


def _xla_erfc(x):
    one = _f(1.0)
    ax = jnp.abs(x)
    lt1 = ax < one
    x2 = x * x
    p = x2 * _f(7.85386146e-05)
    p = p + _f(-0.000801019371)
    p = p * x2 + _f(0.00518832775)
    p = p * x2 + _f(-0.0268538129)
    p = p * x2 + _f(0.112835854)
    p = p * x2 + _f(-0.37612626)
    p = p * x2 + _f(1.12837911)
    erf_branch = one - x * p

    nx2 = -x2
    underflow = nx2 < _f(-88.7228394)
    z = jnp.exp(nx2)
    zq = z * (one / ax)
    r = one / x2
    p1 = r * _f(0.0232682)
    p1 = p1 + _f(-0.138703942)
    p1 = p1 * r + _f(0.368742466)
    p1 = p1 * r + _f(-0.582473278)
    p1 = p1 * r + _f(0.621000469)
    p1 = p1 * r + _f(-0.494451523)
    p1 = p1 * r + _f(0.340488)
    p1 = p1 * r + _f(-0.274112701)
    p1 = p1 * r + _f(0.563825965)
    p2 = r * _f(-10.477664)
    p2 = p2 + _f(12.9772)
    p2 = p2 * r + _f(-7.49551868)
    p2 = p2 * r + _f(2.92101908)
    p2 = p2 * r + _f(-1.01526523)
    p2 = p2 * r + _f(0.42184633)
    p2 = p2 * r + _f(-0.282076746)
    p2 = p2 * r + _f(0.564189494)
    poly = jnp.where(ax < _f(2.0), p1, p2)
    y = zq * poly
    y = jnp.where(underflow, _f(0.0), y)
    y = jnp.where(x < _f(0.0), _f(2.0) - y, y)
    return jnp.where(lt1, erf_branch, y)


def _xla_gelu(x):
    return (_f(0.5) * x) * _xla_erfc((-x) * _f(0.7071067690849304))


def _score_body(x_ref, w1_ref, b1_ref, w2_ref, b2_ref,
                out_ref, t_ref, need_ref, keys_scr):
    i = pl.program_id(0)
    HB = BLK // 2
    w1 = w1_ref[...]
    w2 = w2_ref[...]
    for t in range(2):
        xb = x_ref[pl.ds(t * HB, HB), :]
        h = jnp.dot(xb, w1, preferred_element_type=jnp.float32)
        h = h + b1_ref[0:1, :]
        g = _xla_gelu(h)
        s = jnp.dot(g, w2, preferred_element_type=jnp.float32)
        s = s[:, 0] + b2_ref[0, 0]
        bits = lax.bitcast_convert_type(s, jnp.uint32)
        key = jnp.where(s >= 0.0, bits | jnp.uint32(0x80000000), ~bits)
        out_ref[0, 0, pl.ds(t * HB, HB)] = key
        keys_scr[pl.ds(i, 1), pl.ds(t * HB, HB)] = key.reshape(1, HB)

    @pl.when(i == NBLK - 1)
    def _select():
        ku = keys_scr[...].reshape(B, S)
        p = jnp.zeros((B, 1), jnp.uint32)
        for j in range(31, -1, -1):
            cand = p | (jnp.uint32(1) << j)
            cnt = jnp.sum((ku >= cand).astype(jnp.int32), axis=1,
                          keepdims=True)
            p = jnp.where(cnt >= K, cand, p)
        cnt_gt = jnp.sum((ku > p).astype(jnp.int32), axis=1, keepdims=True)
        need = (K - cnt_gt).astype(jnp.int32)
        pad = jnp.zeros((8 - B, 1), p.dtype)
        t_ref[...] = jnp.broadcast_to(
            jnp.concatenate([p, pad], axis=0), (8, 128))
        padn = jnp.zeros((8 - B, 1), need.dtype)
        need_ref[...] = jnp.broadcast_to(
            jnp.concatenate([need, padn], axis=0), (8, 128))


def _select_body(keys_ref, t_ref, need_ref):
    ku = keys_ref[...]
    p = jnp.zeros((B, 1), jnp.uint32)
    for j in range(31, -1, -1):
        cand = p | (jnp.uint32(1) << j)
        cnt = jnp.sum((ku >= cand).astype(jnp.int32), axis=1, keepdims=True)
        p = jnp.where(cnt >= K, cand, p)
    cnt_gt = jnp.sum((ku > p).astype(jnp.int32), axis=1, keepdims=True)
    need = (K - cnt_gt).astype(jnp.int32)
    pad = jnp.zeros((8 - B, 1), p.dtype)
    t_ref[...] = jnp.broadcast_to(jnp.concatenate([p, pad], axis=0), (8, 128))
    padn = jnp.zeros((8 - B, 1), need.dtype)
    need_ref[...] = jnp.broadcast_to(
        jnp.concatenate([need, padn], axis=0), (8, 128))


CH = S // TPB


def _sc_body(x_hbm, keys_hbm, t_hbm, need_hbm, out_hbm,
             keys_v, t_v, need_v, idxs_l, cnt_st, cnt_v, staged_v, my_v,
             rows_v, shr_cnt, shr_idx, gsem0, gsem1, wsem0, wsem1):
    cid = lax.axis_index("c")
    sid = lax.axis_index("s")
    wid = cid * NS + sid
    b = wid // TPB
    c0 = (wid % TPB) * CH
    lanes = lax.iota(jnp.int32, L)
    one = jnp.int32(1)
    zero = jnp.int32(0)

    pltpu.sync_copy(keys_hbm.at[pl.ds(b * S + c0, CH)], keys_v)
    pltpu.sync_copy(t_hbm.at[pl.ds(b * 128, L)], t_v)
    pltpu.sync_copy(need_hbm.at[pl.ds(b * 128, L)], need_v)
    tv = t_v[...]
    nv = need_v[...]

    def stepA(i, carry):
        ag, ae = carry
        kv = keys_v[pl.ds(i * L, L)]
        return (ag + jnp.where(kv > tv, one, zero),
                ae + jnp.where(kv == tv, one, zero))

    zv = jnp.zeros((L,), jnp.int32)
    ag, ae = lax.fori_loop(0, CH // L, stepA, (zv, zv))
    gt_cnt = jnp.sum(ag)
    eq_cnt = jnp.sum(ae)
    cnt_st[...] = jnp.where(lanes == 0, gt_cnt,
                            jnp.where(lanes == 1, eq_cnt, zero))
    pltpu.sync_copy(cnt_st, shr_cnt.at[sid])
    plsc.subcore_barrier()

    pltpu.sync_copy(shr_cnt, cnt_v)
    gtv = plsc.load_gather(cnt_v, [lanes, jnp.zeros((L,), jnp.int32)])
    eqv = plsc.load_gather(cnt_v, [lanes, jnp.ones((L,), jnp.int32)])
    g0 = (sid // TPB) * TPB
    in_group = jnp.logical_and(lanes >= g0, lanes < g0 + TPB)
    eqv_g = jnp.where(in_group, eqv, zero)
    ecs_g = plsc.cumsum(eqv_g)
    eq_before_v = ecs_g - eqv_g
    eqkeep = jnp.minimum(jnp.maximum(nv - eq_before_v, zero), eqv)
    kept_v = jnp.where(in_group, gtv + eqkeep, zero)
    pcs = plsc.cumsum(kept_v)
    p_excl = pcs - kept_v
    my_eq_before = jnp.sum(jnp.where(lanes == sid, eq_before_v, zero))

    base = jnp.int32(b * S) + c0

    def stepB(i, carry):
        off, eq_seen = carry
        kv = keys_v[pl.ds(i * L, L)]
        gt = kv > tv
        eq = kv == tv
        eqi = jnp.where(eq, one, zero)
        ecs = plsc.cumsum(eqi)
        keep_eq = jnp.logical_and(eq, (ecs + eq_seen) <= nv)
        m = jnp.logical_or(gt, keep_eq)
        mi = jnp.where(m, one, zero)
        mcs = plsc.cumsum(mi)
        pos = off + (mcs - mi)
        idxv = lanes + (i * L + base)
        plsc.store_scatter(idxs_l, [pos], idxv, mask=m)
        return off + jnp.max(mcs), eq_seen + jnp.max(ecs)

    lax.fori_loop(0, CH // L, stepB, (zero, my_eq_before))
    pltpu.sync_copy(idxs_l.at[pl.ds(0, CH)], shr_idx.at[sid])
    plsc.subcore_barrier()

    lo = (wid % TPB) * ROWS_PER_TILE
    pltpu.sync_copy(shr_idx.at[pl.ds(g0, TPB)], staged_v)
    p_scal = [jnp.sum(jnp.where(lanes == g0 + r, p_excl, zero))
              for r in range(TPB)]
    for j in range(ROWS_PER_TILE // L):
        rankv = lanes + (lo + j * L)
        srcv = jnp.full((L,), -1, jnp.int32)
        basev = jnp.zeros((L,), jnp.int32)
        for r in range(TPB):
            hit = rankv >= p_scal[r]
            srcv = srcv + jnp.where(hit, one, zero)
            basev = jnp.maximum(basev, jnp.where(hit, p_scal[r], zero))
        vals = plsc.load_gather(staged_v, [srcv, rankv - basev])
        my_v[pl.ds(j * L, L)] = vals
    nch = ROWS_PER_TILE // GCH
    gsems = (gsem0, gsem1)
    wsems = (wsem0, wsem1)
    bufs = (rows_v.at[0], rows_v.at[1])
    for j in range(2):
        pltpu.async_copy(
            x_hbm.at[my_v.at[pl.ds(j * GCH, GCH)]], bufs[j], gsems[j])
    for j in range(nch):
        bi = j % 2
        pltpu.make_async_copy(
            x_hbm.at[my_v.at[pl.ds(j * GCH, GCH)]], bufs[bi], gsems[bi]).wait()
        dst = out_hbm.at[pl.ds(b * K + lo + j * GCH, GCH), :]
        w = pltpu.async_copy(bufs[bi], dst, wsems[bi])
        if j + 2 < nch:
            w.wait()
            pltpu.async_copy(
                x_hbm.at[my_v.at[pl.ds((j + 2) * GCH, GCH)]], bufs[bi],
                gsems[bi])
        else:
            w.wait()


def _compact_gather(x2d, keys_flat, t_flat, need_flat):
    mesh = plsc.VectorSubcoreMesh(core_axis_name="c", subcore_axis_name="s")
    kern = pl.kernel(
        _sc_body,
        out_type=jax.ShapeDtypeStruct((B * K, H), jnp.float32),
        mesh=mesh,
        compiler_params=pltpu.CompilerParams(needs_layout_passes=False),
        scratch_types=[
            pltpu.VMEM((CH,), jnp.uint32),
            pltpu.VMEM((L,), jnp.uint32),
            pltpu.VMEM((L,), jnp.int32),
            pltpu.VMEM((CH + L,), jnp.int32),
            pltpu.VMEM((L,), jnp.int32),
            pltpu.VMEM((NS, L), jnp.int32),
            pltpu.VMEM((TPB, CH), jnp.int32),
            pltpu.VMEM((ROWS_PER_TILE,), jnp.int32),
            pltpu.VMEM((2, GCH, H), jnp.float32),
            pltpu.VMEM_SHARED((NS, L), jnp.int32),
            pltpu.VMEM_SHARED((NS, CH), jnp.int32),
            pltpu.SemaphoreType.DMA,
            pltpu.SemaphoreType.DMA,
            pltpu.SemaphoreType.DMA,
            pltpu.SemaphoreType.DMA,
        ],
    )
    return kern(x2d, keys_flat, t_flat, need_flat)


@jax.jit
def kernel(x, W1, b1, W2, b2):
    x2d = x.reshape(B * S, H)
    b1r = jnp.broadcast_to(b1[None, :], (8, HQ))
    w2p = jnp.pad(W2, ((0, 0), (0, 128 - W2.shape[1])))
    b2r = jnp.broadcast_to(b2[None, :], (8, 128))

    keys3, t8, need8 = pl.pallas_call(
        _score_body,
        grid=(NBLK,),
        in_specs=[
            pl.BlockSpec((BLK, H), lambda i: (i, 0)),
            pl.BlockSpec((H, HQ), lambda i: (0, 0)),
            pl.BlockSpec((8, HQ), lambda i: (0, 0)),
            pl.BlockSpec((HQ, 128), lambda i: (0, 0)),
            pl.BlockSpec((8, 128), lambda i: (0, 0)),
        ],
        out_specs=(
            pl.BlockSpec((1, 1, BLK), lambda i: (i, 0, 0)),
            pl.BlockSpec((8, 128), lambda i: (0, 0)),
            pl.BlockSpec((8, 128), lambda i: (0, 0)),
        ),
        out_shape=(
            jax.ShapeDtypeStruct((NBLK, 1, BLK), jnp.uint32),
            jax.ShapeDtypeStruct((8, 128), jnp.uint32),
            jax.ShapeDtypeStruct((8, 128), jnp.int32),
        ),
        scratch_shapes=[pltpu.VMEM((NBLK, BLK), jnp.uint32)],
    )(x2d, W1, b1r, w2p, b2r)
    keys = keys3.reshape(B, S)

    out_flat = _compact_gather(
        x2d, keys.reshape(-1), t8.reshape(-1), need8.reshape(-1))
    return out_flat.reshape(B, K, H)
